# Initial kernel scaffold; baseline (speedup 1.0000x reference)
import jax
import jax.numpy as jnp
from jax.experimental import pallas as pl


def kernel(x, emb_table, lin_table, lin_bias, W1, b1, g1, be1, W2, b2, g2, be2, W3, b3):
    raise NotImplementedError("write your pallas kernel here")


# speedup vs baseline: 1.7078x; 1.7078x over previous
import functools

import jax
import jax.numpy as jnp
import numpy as np
from jax import lax
from jax.experimental import pallas as pl
from jax.experimental.pallas import tpu as pltpu
from jax.experimental.pallas import tpu_sc as plsc

_B = 16384
_NF = 26
_ED = 16
_H1 = 256
_H2 = 128
_EPS = 1e-5
_NUM_TABLE_ROWS = 26 * 100000
_OFFSETS = (np.arange(26, dtype=np.int32) * 100000)

_NC = 2
_NS = 16
_NW = _NC * _NS
_IDXW = (_B * _NF) // _NW
_CH = 128
_NCHUNK = _IDXW // _CH


_SLAB = 8
_SLABIDX = _SLAB * _CH
_NSLAB = _IDXW // _SLABIDX


def _sc_gather_body(xo_hbm, embt_hbm, lint_hbm, eout_hbm, lout_hbm,
                    idx_v, erows_v, lrows_v, sem_g, sem_w):
    c = lax.axis_index("c")
    s = lax.axis_index("s")
    wid = s * _NC + c
    base = wid * _IDXW
    pltpu.sync_copy(xo_hbm.at[pl.ds(base, _IDXW)], idx_v)

    def body(k, carry):
        off0 = k * _SLABIDX
        copies = []
        for j in range(_SLAB):
            idx_sl = idx_v.at[pl.ds(off0 + j * _CH, _CH)]
            copies.append(pltpu.async_copy(
                embt_hbm.at[idx_sl], erows_v.at[pl.ds(j * _CH, _CH)], sem_g))
            copies.append(pltpu.async_copy(
                lint_hbm.at[idx_sl], lrows_v.at[pl.ds(j * _CH, _CH)], sem_g))
        for cp in copies:
            cp.wait()
        we = pltpu.async_copy(erows_v, eout_hbm.at[pl.ds(base + off0, _SLABIDX)], sem_w)
        wl = pltpu.async_copy(lrows_v, lout_hbm.at[pl.ds(base + off0, _SLABIDX)], sem_w)
        we.wait()
        wl.wait()
        return carry

    lax.fori_loop(0, _NSLAB, body, 0)


@functools.lru_cache(maxsize=1)
def _sc_gather():
    return pl.kernel(
        _sc_gather_body,
        mesh=plsc.VectorSubcoreMesh(core_axis_name="c", subcore_axis_name="s",
                                    num_cores=_NC, num_subcores=_NS),
        out_type=[
            jax.ShapeDtypeStruct((_B * _NF, _ED), jnp.float32),
            jax.ShapeDtypeStruct((_B * _NF, 1), jnp.float32),
        ],
        scratch_types=[
            pltpu.VMEM((_IDXW,), jnp.int32),
            pltpu.VMEM((_SLABIDX, _ED), jnp.float32),
            pltpu.VMEM((_SLABIDX, 1), jnp.float32),
            pltpu.SemaphoreType.DMA,
            pltpu.SemaphoreType.DMA,
        ],
        compiler_params=pltpu.CompilerParams(use_tc_tiling_on_sc=False),
    )


_BT = 1024


def _mlp_body(emb_ref, lin_ref, w1_ref, b1_ref, g1_ref, be1_ref,
              w2_ref, b2_ref, g2_ref, be2_ref, w3t_ref, bias_ref, out_ref):
    r = float(1.0 / np.sqrt(1.0 + _EPS))
    s1 = g1_ref[...] * r
    s2 = g2_ref[...] * r
    h = jnp.dot(emb_ref[...], w1_ref[...], preferred_element_type=jnp.float32)
    h = h * s1 + (b1_ref[...] * s1 + be1_ref[...])
    h = jnp.maximum(h, 0.0)
    h = jnp.dot(h, w2_ref[...], preferred_element_type=jnp.float32)
    h = h * s2 + (b2_ref[...] * s2 + be2_ref[...])
    h = jnp.maximum(h, 0.0)
    deep = jnp.sum(h * w3t_ref[...], axis=1, keepdims=True)
    linsum = jnp.sum(lin_ref[...], axis=1, keepdims=True)
    out_ref[...] = deep + linsum + bias_ref[0, 0]


def _mlp_call(emb, lin, w1, b1, g1, be1, w2, b2, g2, be2, w3t, bias):
    full = lambda shape: pl.BlockSpec(shape, lambda i: (0, 0))
    return pl.pallas_call(
        _mlp_body,
        grid=(_B // _BT,),
        in_specs=[
            pl.BlockSpec((_BT, _NF * _ED), lambda i: (i, 0)),
            pl.BlockSpec((_BT, _NF), lambda i: (i, 0)),
            full((_NF * _ED, _H1)),
            full((1, _H1)),
            full((1, _H1)),
            full((1, _H1)),
            full((_H1, _H2)),
            full((1, _H2)),
            full((1, _H2)),
            full((1, _H2)),
            full((1, _H2)),
            full((1, 1)),
        ],
        out_specs=pl.BlockSpec((_BT, 1), lambda i: (i, 0)),
        out_shape=jax.ShapeDtypeStruct((_B, 1), jnp.float32),
    )(emb, lin, w1, b1, g1, be1, w2, b2, g2, be2, w3t, bias)


def kernel(x, emb_table, lin_table, lin_bias, W1, b1, g1, be1,
           W2, b2, g2, be2, W3, b3):
    xo = (x.astype(jnp.int32) + _OFFSETS[None, :]).reshape(-1)
    erows, lrows = _sc_gather()(xo, emb_table, lin_table)
    emb = erows.reshape(_B, _NF * _ED)
    lin = lrows.reshape(_B, _NF)
    bias = (lin_bias[0] + b3[0]).reshape(1, 1)
    out = _mlp_call(
        emb, lin, W1,
        b1.reshape(1, _H1), g1.reshape(1, _H1), be1.reshape(1, _H1),
        W2, b2.reshape(1, _H2), g2.reshape(1, _H2), be2.reshape(1, _H2),
        W3.reshape(1, _H2), bias)
    return out


# speedup vs baseline: 7.2465x; 4.2431x over previous
import functools

import jax
import jax.numpy as jnp
import numpy as np
from jax import lax
from jax.experimental import pallas as pl
from jax.experimental.pallas import tpu as pltpu
from jax.experimental.pallas import tpu_sc as plsc

_B = 16384
_NF = 26
_ED = 16
_H1 = 256
_H2 = 128
_EPS = 1e-5
_NUM_TABLE_ROWS = 26 * 100000
_OFFSETS = (np.arange(26, dtype=np.int32) * 100000)

_NC = 2
_NS = 16
_NW = _NC * _NS
_IDXW = (_B * _NF) // _NW
_CH = 128
_NCHUNK = _IDXW // _CH


_SLAB = 8
_SLABIDX = _SLAB * _CH
_NSLAB = _IDXW // _SLABIDX


def _sc_gather_body(xo_hbm, embt_hbm, lint_hbm, eout_hbm, lout_hbm,
                    idx_v, erows_v, lrows_v, sem_g, sem_w):
    c = lax.axis_index("c")
    s = lax.axis_index("s")
    wid = s * _NC + c
    base = wid * _IDXW
    pltpu.sync_copy(xo_hbm.at[pl.ds(base, _IDXW)], idx_v)

    def body(k, carry):
        off0 = k * _SLABIDX
        copies = []
        for j in range(_SLAB):
            idx_sl = idx_v.at[pl.ds(off0 + j * _CH, _CH)]
            copies.append(pltpu.async_copy(
                embt_hbm.at[idx_sl], erows_v.at[pl.ds(j * _CH, _CH)], sem_g))
            copies.append(pltpu.async_copy(
                lint_hbm.at[idx_sl], lrows_v.at[pl.ds(j * _CH, _CH)], sem_g))
        for cp in copies:
            cp.wait()
        we = pltpu.async_copy(erows_v, eout_hbm.at[pl.ds(base + off0, _SLABIDX)], sem_w)
        wl = pltpu.async_copy(lrows_v, lout_hbm.at[pl.ds(base + off0, _SLABIDX)], sem_w)
        we.wait()
        wl.wait()
        return carry

    lax.fori_loop(0, _NSLAB, body, 0)


@functools.lru_cache(maxsize=1)
def _sc_gather():
    return pl.kernel(
        _sc_gather_body,
        mesh=plsc.VectorSubcoreMesh(core_axis_name="c", subcore_axis_name="s",
                                    num_cores=_NC, num_subcores=_NS),
        out_type=[
            jax.ShapeDtypeStruct((_B * _NF, _ED), jnp.float32),
            jax.ShapeDtypeStruct((_B * _NF,), jnp.float32),
        ],
        scratch_types=[
            pltpu.VMEM((_IDXW,), jnp.int32),
            pltpu.VMEM((_SLABIDX, _ED), jnp.float32),
            pltpu.VMEM((_SLABIDX,), jnp.float32),
            pltpu.SemaphoreType.DMA,
            pltpu.SemaphoreType.DMA,
        ],
        compiler_params=pltpu.CompilerParams(use_tc_tiling_on_sc=False),
    )


_BT = 1024


def _mlp_body(emb_ref, lin_ref, w1_ref, b1_ref, g1_ref, be1_ref,
              w2_ref, b2_ref, g2_ref, be2_ref, w3t_ref, bias_ref, out_ref):
    r = float(1.0 / np.sqrt(1.0 + _EPS))
    s1 = g1_ref[...] * r
    s2 = g2_ref[...] * r
    h = jnp.dot(emb_ref[...], w1_ref[...], preferred_element_type=jnp.float32)
    h = h * s1 + (b1_ref[...] * s1 + be1_ref[...])
    h = jnp.maximum(h, 0.0)
    h = jnp.dot(h, w2_ref[...], preferred_element_type=jnp.float32)
    h = h * s2 + (b2_ref[...] * s2 + be2_ref[...])
    h = jnp.maximum(h, 0.0)
    deep = jnp.sum(h * w3t_ref[...], axis=1, keepdims=True)
    linsum = jnp.sum(lin_ref[...], axis=1, keepdims=True)
    out_ref[...] = deep + linsum + bias_ref[0, 0]


def _mlp_call(emb, lin, w1, b1, g1, be1, w2, b2, g2, be2, w3t, bias):
    full = lambda shape: pl.BlockSpec(shape, lambda i: (0, 0))
    return pl.pallas_call(
        _mlp_body,
        grid=(_B // _BT,),
        in_specs=[
            pl.BlockSpec((_BT, _NF * _ED), lambda i: (i, 0)),
            pl.BlockSpec((_BT, _NF), lambda i: (i, 0)),
            full((_NF * _ED, _H1)),
            full((1, _H1)),
            full((1, _H1)),
            full((1, _H1)),
            full((_H1, _H2)),
            full((1, _H2)),
            full((1, _H2)),
            full((1, _H2)),
            full((1, _H2)),
            full((1, 1)),
        ],
        out_specs=pl.BlockSpec((_BT, 1), lambda i: (i, 0)),
        out_shape=jax.ShapeDtypeStruct((_B, 1), jnp.float32),
    )(emb, lin, w1, b1, g1, be1, w2, b2, g2, be2, w3t, bias)


def kernel(x, emb_table, lin_table, lin_bias, W1, b1, g1, be1,
           W2, b2, g2, be2, W3, b3):
    xo = (x.astype(jnp.int32) + _OFFSETS[None, :]).reshape(-1)
    erows, lrows = _sc_gather()(xo, emb_table, lin_table.reshape(-1))
    emb = erows.reshape(_B, _NF * _ED)
    lin = lrows.reshape(_B, _NF)
    bias = (lin_bias[0] + b3[0]).reshape(1, 1)
    out = _mlp_call(
        emb, lin, W1,
        b1.reshape(1, _H1), g1.reshape(1, _H1), be1.reshape(1, _H1),
        W2, b2.reshape(1, _H2), g2.reshape(1, _H2), be2.reshape(1, _H2),
        W3.reshape(1, _H2), bias)
    return out


# speedup vs baseline: 7.3319x; 1.0118x over previous
import functools

import jax
import jax.numpy as jnp
import numpy as np
from jax import lax
from jax.experimental import pallas as pl
from jax.experimental.pallas import tpu as pltpu
from jax.experimental.pallas import tpu_sc as plsc

_B = 16384
_NF = 26
_ED = 16
_H1 = 256
_H2 = 128
_EPS = 1e-5
_NUM_TABLE_ROWS = 26 * 100000
_OFFSETS = (np.arange(26, dtype=np.int32) * 100000)

_NC = 2
_NS = 16
_NW = _NC * _NS
_IDXW = (_B * _NF) // _NW
_CH = 128
_NCHUNK = _IDXW // _CH


_SLAB = 8
_SLABIDX = _SLAB * _CH
_NSLAB = _IDXW // _SLABIDX


def _sc_gather_body(xo_hbm, embt_hbm, lint_hbm, eout_hbm, lout_hbm,
                    idx_v, erows_v, lrows_v, sem_g, sem_w):
    c = lax.axis_index("c")
    s = lax.axis_index("s")
    wid = s * _NC + c
    base = wid * _IDXW
    pltpu.sync_copy(xo_hbm.at[pl.ds(base, _IDXW)], idx_v)

    def body(k, carry):
        off0 = k * _SLABIDX
        copies = []
        for j in range(_SLAB):
            idx_sl = idx_v.at[pl.ds(off0 + j * _CH, _CH)]
            copies.append(pltpu.async_copy(
                embt_hbm.at[idx_sl], erows_v.at[pl.ds(j * _CH, _CH)], sem_g))
            copies.append(pltpu.async_copy(
                lint_hbm.at[idx_sl], lrows_v.at[pl.ds(j * _CH, _CH)], sem_g))
        for cp in copies:
            cp.wait()
        we = pltpu.async_copy(erows_v, eout_hbm.at[pl.ds(base + off0, _SLABIDX)], sem_w)
        wl = pltpu.async_copy(lrows_v, lout_hbm.at[pl.ds(base + off0, _SLABIDX)], sem_w)
        we.wait()
        wl.wait()
        return carry

    lax.fori_loop(0, _NSLAB, body, 0)


@functools.lru_cache(maxsize=1)
def _sc_gather():
    return pl.kernel(
        _sc_gather_body,
        mesh=plsc.VectorSubcoreMesh(core_axis_name="c", subcore_axis_name="s",
                                    num_cores=_NC, num_subcores=_NS),
        out_type=[
            jax.ShapeDtypeStruct((_B * _NF, _ED), jnp.float32),
            jax.ShapeDtypeStruct((_B * _NF,), jnp.float32),
        ],
        scratch_types=[
            pltpu.VMEM((_IDXW,), jnp.int32),
            pltpu.VMEM((_SLABIDX, _ED), jnp.float32),
            pltpu.VMEM((_SLABIDX,), jnp.float32),
            pltpu.SemaphoreType.DMA,
            pltpu.SemaphoreType.DMA,
        ],
        compiler_params=pltpu.CompilerParams(use_tc_tiling_on_sc=False),
    )


_NTILE = _NUM_TABLE_ROWS // 128
_TAILW = _NUM_TABLE_ROWS - _NTILE * 128
_NGRP = _NTILE // 8
_NPAIR = _NGRP // 2
_PAIR_BASE = _NPAIR // _NW
_PAIR_EXTRA = _NPAIR - _PAIR_BASE * _NW


def _cvt_body(embt_hbm, tail_hbm, out_hbm, st0, st1, ob0, ob1, si0, si1):
    c_ = lax.axis_index("c")
    s_ = lax.axis_index("s")
    wid = s_ * _NC + c_
    npair = _PAIR_BASE + (wid < _PAIR_EXTRA).astype(jnp.int32)
    pstart = wid * _PAIR_BASE + lax.min(wid, _PAIR_EXTRA)
    iota = lax.broadcasted_iota(jnp.int32, (16,), 0)

    def fire(g, st, si):
        return [pltpu.async_copy(
            embt_hbm.at[:, pl.ds((g * 8 + j) * 128, 128)], st.at[j], si)
            for j in range(8)]

    def permute(st, ob):
        for j in range(8):
            jvec = iota * 0 + j

            def cbody(c, carry):
                v = plsc.load_gather(st, [jvec, iota, iota * 0 + c])
                ob[pl.ds(j * 2048 + c * 16, 16)] = v
                return carry

            lax.fori_loop(0, 128, cbody, 0)

    def pbody(k, carry):
        g0 = (pstart + k) * 2
        g1 = g0 + 1
        cp0 = fire(g0, st0, si0)
        cp1 = fire(g1, st1, si1)
        for cp in cp0:
            cp.wait()
        permute(st0, ob0)
        pltpu.sync_copy(ob0, out_hbm.at[pl.ds(g0 * 16384, 16384)])
        for cp in cp1:
            cp.wait()
        permute(st1, ob1)
        pltpu.sync_copy(ob1, out_hbm.at[pl.ds(g1 * 16384, 16384)])
        return carry

    lax.fori_loop(0, npair, pbody, 0)

    @pl.when(wid == _NW - 1)
    def _leftovers():
        gl = _NGRP - 1
        for cp in fire(gl, st0, si0):
            cp.wait()
        permute(st0, ob0)
        pltpu.sync_copy(ob0, out_hbm.at[pl.ds(gl * 16384, 16384)])
        pltpu.sync_copy(tail_hbm, ob1.at[pl.ds(0, _TAILW * _ED)])
        pltpu.sync_copy(ob1.at[pl.ds(0, _TAILW * _ED)],
                        out_hbm.at[pl.ds(_NTILE * 2048, _TAILW * _ED)])


@functools.lru_cache(maxsize=1)
def _cvt_call():
    return pl.kernel(
        _cvt_body,
        mesh=plsc.VectorSubcoreMesh(core_axis_name="c", subcore_axis_name="s",
                                    num_cores=_NC, num_subcores=_NS),
        out_type=jax.ShapeDtypeStruct((_NUM_TABLE_ROWS * _ED,), jnp.float32),
        scratch_types=[
            pltpu.VMEM((8, _ED, 128), jnp.float32),
            pltpu.VMEM((8, _ED, 128), jnp.float32),
            pltpu.VMEM((16384,), jnp.float32),
            pltpu.VMEM((16384,), jnp.float32),
            pltpu.SemaphoreType.DMA,
            pltpu.SemaphoreType.DMA,
        ],
        compiler_params=pltpu.CompilerParams(use_tc_tiling_on_sc=True,
                                             needs_layout_passes=False),
    )


_BT = 1024


def _mlp_body(emb_ref, lin_ref, w1_ref, b1_ref, g1_ref, be1_ref,
              w2_ref, b2_ref, g2_ref, be2_ref, w3t_ref, bias_ref, out_ref):
    r = float(1.0 / np.sqrt(1.0 + _EPS))
    s1 = g1_ref[...] * r
    s2 = g2_ref[...] * r
    h = jnp.dot(emb_ref[...], w1_ref[...], preferred_element_type=jnp.float32)
    h = h * s1 + (b1_ref[...] * s1 + be1_ref[...])
    h = jnp.maximum(h, 0.0)
    h = jnp.dot(h, w2_ref[...], preferred_element_type=jnp.float32)
    h = h * s2 + (b2_ref[...] * s2 + be2_ref[...])
    h = jnp.maximum(h, 0.0)
    deep = jnp.sum(h * w3t_ref[...], axis=1, keepdims=True)
    linsum = jnp.sum(lin_ref[...], axis=1, keepdims=True)
    out_ref[...] = deep + linsum + bias_ref[0, 0]


def _mlp_call(emb, lin, w1, b1, g1, be1, w2, b2, g2, be2, w3t, bias):
    full = lambda shape: pl.BlockSpec(shape, lambda i: (0, 0))
    return pl.pallas_call(
        _mlp_body,
        grid=(_B // _BT,),
        in_specs=[
            pl.BlockSpec((_BT, _NF * _ED), lambda i: (i, 0)),
            pl.BlockSpec((_BT, _NF), lambda i: (i, 0)),
            full((_NF * _ED, _H1)),
            full((1, _H1)),
            full((1, _H1)),
            full((1, _H1)),
            full((_H1, _H2)),
            full((1, _H2)),
            full((1, _H2)),
            full((1, _H2)),
            full((1, _H2)),
            full((1, 1)),
        ],
        out_specs=pl.BlockSpec((_BT, 1), lambda i: (i, 0)),
        out_shape=jax.ShapeDtypeStruct((_B, 1), jnp.float32),
    )(emb, lin, w1, b1, g1, be1, w2, b2, g2, be2, w3t, bias)


def kernel(x, emb_table, lin_table, lin_bias, W1, b1, g1, be1,
           W2, b2, g2, be2, W3, b3):
    xo = (x.astype(jnp.int32) + _OFFSETS[None, :]).reshape(-1)
    tail = emb_table[_NTILE * 128:].reshape(-1)
    table_rm = _cvt_call()(emb_table.T, tail).reshape(_NUM_TABLE_ROWS, _ED)
    erows, lrows = _sc_gather()(xo, table_rm, lin_table.reshape(-1))
    emb = erows.reshape(_B, _NF * _ED)
    lin = lrows.reshape(_B, _NF)
    bias = (lin_bias[0] + b3[0]).reshape(1, 1)
    out = _mlp_call(
        emb, lin, W1,
        b1.reshape(1, _H1), g1.reshape(1, _H1), be1.reshape(1, _H1),
        W2, b2.reshape(1, _H2), g2.reshape(1, _H2), be2.reshape(1, _H2),
        W3.reshape(1, _H2), bias)
    return out


# speedup vs baseline: 11.2874x; 1.5395x over previous
import functools

import jax
import jax.numpy as jnp
import numpy as np
from jax import lax
from jax.experimental import pallas as pl
from jax.experimental.pallas import tpu as pltpu
from jax.experimental.pallas import tpu_sc as plsc

_B = 16384
_NF = 26
_ED = 16
_H1 = 256
_H2 = 128
_EPS = 1e-5
_NUM_TABLE_ROWS = 26 * 100000
_OFFSETS = (np.arange(26, dtype=np.int32) * 100000)

_NC = 2
_NS = 16
_NW = _NC * _NS
_IDXW = (_B * _NF) // _NW
_CH = 128
_NCHUNK = _IDXW // _CH


_SLAB = 8
_SLABIDX = _SLAB * _CH
_NSLAB = _IDXW // _SLABIDX


def _sc_gather_body(xo_hbm, embt_hbm, lint_hbm, eout_hbm, lout_hbm,
                    idx_v, erows_v, lrows_v, sem_g, sem_w):
    c = lax.axis_index("c")
    s = lax.axis_index("s")
    wid = s * _NC + c
    base = wid * _IDXW
    pltpu.sync_copy(xo_hbm.at[pl.ds(base, _IDXW)], idx_v)

    def body(k, carry):
        off0 = k * _SLABIDX
        copies = []
        for j in range(_SLAB):
            idx_sl = idx_v.at[pl.ds(off0 + j * _CH, _CH)]
            copies.append(pltpu.async_copy(
                embt_hbm.at[idx_sl], erows_v.at[pl.ds(j * _CH, _CH)], sem_g))
            copies.append(pltpu.async_copy(
                lint_hbm.at[idx_sl], lrows_v.at[pl.ds(j * _CH, _CH)], sem_g))
        for cp in copies:
            cp.wait()
        we = pltpu.async_copy(erows_v, eout_hbm.at[pl.ds(base + off0, _SLABIDX)], sem_w)
        wl = pltpu.async_copy(lrows_v, lout_hbm.at[pl.ds(base + off0, _SLABIDX)], sem_w)
        we.wait()
        wl.wait()
        return carry

    lax.fori_loop(0, _NSLAB, body, 0)


@functools.lru_cache(maxsize=1)
def _sc_gather():
    return pl.kernel(
        _sc_gather_body,
        mesh=plsc.VectorSubcoreMesh(core_axis_name="c", subcore_axis_name="s",
                                    num_cores=_NC, num_subcores=_NS),
        out_type=[
            jax.ShapeDtypeStruct((_B * _NF, _ED), jnp.float32),
            jax.ShapeDtypeStruct((_B * _NF,), jnp.float32),
        ],
        scratch_types=[
            pltpu.VMEM((_IDXW,), jnp.int32),
            pltpu.VMEM((_SLABIDX, _ED), jnp.float32),
            pltpu.VMEM((_SLABIDX,), jnp.float32),
            pltpu.SemaphoreType.DMA,
            pltpu.SemaphoreType.DMA,
        ],
        compiler_params=pltpu.CompilerParams(use_tc_tiling_on_sc=False),
    )


_NTILE = _NUM_TABLE_ROWS // 128
_TAILW = _NUM_TABLE_ROWS - _NTILE * 128
_NGRP = _NTILE // 8
_NPAIR = _NGRP // 2
_PAIR_BASE = _NPAIR // _NW
_PAIR_EXTRA = _NPAIR - _PAIR_BASE * _NW


def _cvt_body(embt_hbm, tail_hbm, out_hbm, st0, st1, ob0, ob1, si0, si1):
    c_ = lax.axis_index("c")
    s_ = lax.axis_index("s")
    wid = s_ * _NC + c_
    npair = _PAIR_BASE + (wid < _PAIR_EXTRA).astype(jnp.int32)
    pstart = wid * _PAIR_BASE + lax.min(wid, _PAIR_EXTRA)
    iota = lax.broadcasted_iota(jnp.int32, (16,), 0)

    def fire(g, st, si):
        return [pltpu.async_copy(
            embt_hbm.at[:, pl.ds((g * 8 + j) * 128, 128)], st.at[j], si)
            for j in range(8)]

    def permute(st, ob):
        for j in range(8):
            jvec = iota * 0 + j

            @plsc.parallel_loop(0, 128, unroll=8)
            def _cloop(c):
                v = plsc.load_gather(st, [jvec, iota, iota * 0 + c])
                ob[pl.ds(j * 2048 + c * 16, 16)] = v

    def pbody(k, carry):
        g0 = (pstart + k) * 2
        g1 = g0 + 1
        cp0 = fire(g0, st0, si0)
        cp1 = fire(g1, st1, si1)
        for cp in cp0:
            cp.wait()
        permute(st0, ob0)
        pltpu.sync_copy(ob0, out_hbm.at[pl.ds(g0 * 16384, 16384)])
        for cp in cp1:
            cp.wait()
        permute(st1, ob1)
        pltpu.sync_copy(ob1, out_hbm.at[pl.ds(g1 * 16384, 16384)])
        return carry

    lax.fori_loop(0, npair, pbody, 0)

    @pl.when(wid == _NW - 1)
    def _leftovers():
        gl = _NGRP - 1
        for cp in fire(gl, st0, si0):
            cp.wait()
        permute(st0, ob0)
        pltpu.sync_copy(ob0, out_hbm.at[pl.ds(gl * 16384, 16384)])
        pltpu.sync_copy(tail_hbm, ob1.at[pl.ds(0, _TAILW * _ED)])
        pltpu.sync_copy(ob1.at[pl.ds(0, _TAILW * _ED)],
                        out_hbm.at[pl.ds(_NTILE * 2048, _TAILW * _ED)])


@functools.lru_cache(maxsize=1)
def _cvt_call():
    return pl.kernel(
        _cvt_body,
        mesh=plsc.VectorSubcoreMesh(core_axis_name="c", subcore_axis_name="s",
                                    num_cores=_NC, num_subcores=_NS),
        out_type=jax.ShapeDtypeStruct((_NUM_TABLE_ROWS * _ED,), jnp.float32),
        scratch_types=[
            pltpu.VMEM((8, _ED, 128), jnp.float32),
            pltpu.VMEM((8, _ED, 128), jnp.float32),
            pltpu.VMEM((16384,), jnp.float32),
            pltpu.VMEM((16384,), jnp.float32),
            pltpu.SemaphoreType.DMA,
            pltpu.SemaphoreType.DMA,
        ],
        compiler_params=pltpu.CompilerParams(use_tc_tiling_on_sc=True,
                                             needs_layout_passes=False),
    )


_BT = 1024


def _mlp_body(emb_ref, lin_ref, w1_ref, b1_ref, g1_ref, be1_ref,
              w2_ref, b2_ref, g2_ref, be2_ref, w3t_ref, bias_ref, out_ref):
    r = float(1.0 / np.sqrt(1.0 + _EPS))
    s1 = g1_ref[...] * r
    s2 = g2_ref[...] * r
    h = jnp.dot(emb_ref[...], w1_ref[...], preferred_element_type=jnp.float32)
    h = h * s1 + (b1_ref[...] * s1 + be1_ref[...])
    h = jnp.maximum(h, 0.0)
    h = jnp.dot(h, w2_ref[...], preferred_element_type=jnp.float32)
    h = h * s2 + (b2_ref[...] * s2 + be2_ref[...])
    h = jnp.maximum(h, 0.0)
    deep = jnp.sum(h * w3t_ref[...], axis=1, keepdims=True)
    linsum = jnp.sum(lin_ref[...], axis=1, keepdims=True)
    out_ref[...] = deep + linsum + bias_ref[0, 0]


def _mlp_call(emb, lin, w1, b1, g1, be1, w2, b2, g2, be2, w3t, bias):
    full = lambda shape: pl.BlockSpec(shape, lambda i: (0, 0))
    return pl.pallas_call(
        _mlp_body,
        grid=(_B // _BT,),
        in_specs=[
            pl.BlockSpec((_BT, _NF * _ED), lambda i: (i, 0)),
            pl.BlockSpec((_BT, _NF), lambda i: (i, 0)),
            full((_NF * _ED, _H1)),
            full((1, _H1)),
            full((1, _H1)),
            full((1, _H1)),
            full((_H1, _H2)),
            full((1, _H2)),
            full((1, _H2)),
            full((1, _H2)),
            full((1, _H2)),
            full((1, 1)),
        ],
        out_specs=pl.BlockSpec((_BT, 1), lambda i: (i, 0)),
        out_shape=jax.ShapeDtypeStruct((_B, 1), jnp.float32),
    )(emb, lin, w1, b1, g1, be1, w2, b2, g2, be2, w3t, bias)


def kernel(x, emb_table, lin_table, lin_bias, W1, b1, g1, be1,
           W2, b2, g2, be2, W3, b3):
    xo = (x.astype(jnp.int32) + _OFFSETS[None, :]).reshape(-1)
    tail = emb_table[_NTILE * 128:].reshape(-1)
    table_rm = _cvt_call()(emb_table.T, tail).reshape(_NUM_TABLE_ROWS, _ED)
    erows, lrows = _sc_gather()(xo, table_rm, lin_table.reshape(-1))
    emb = erows.reshape(_B, _NF * _ED)
    lin = lrows.reshape(_B, _NF)
    bias = (lin_bias[0] + b3[0]).reshape(1, 1)
    out = _mlp_call(
        emb, lin, W1,
        b1.reshape(1, _H1), g1.reshape(1, _H1), be1.reshape(1, _H1),
        W2, b2.reshape(1, _H2), g2.reshape(1, _H2), be2.reshape(1, _H2),
        W3.reshape(1, _H2), bias)
    return out


# speedup vs baseline: 11.3557x; 1.0060x over previous
import functools

import jax
import jax.numpy as jnp
import numpy as np
from jax import lax
from jax.experimental import pallas as pl
from jax.experimental.pallas import tpu as pltpu
from jax.experimental.pallas import tpu_sc as plsc

_B = 16384
_NF = 26
_ED = 16
_H1 = 256
_H2 = 128
_EPS = 1e-5
_NUM_TABLE_ROWS = 26 * 100000
_OFFSETS = (np.arange(26, dtype=np.int32) * 100000)

_NC = 2
_NS = 16
_NW = _NC * _NS
_IDXW = (_B * _NF) // _NW
_CH = 128
_NCHUNK = _IDXW // _CH


_SLAB = 8
_SLABIDX = _SLAB * _CH
_NSLAB = _IDXW // _SLABIDX


def _sc_gather_body(xo_hbm, embt_hbm, lint_hbm, eout_hbm, lout_hbm,
                    idx_v, erows_v, lrows_v, sem_g, sem_w):
    c = lax.axis_index("c")
    s = lax.axis_index("s")
    wid = s * _NC + c
    base = wid * _IDXW
    pltpu.sync_copy(xo_hbm.at[pl.ds(base, _IDXW)], idx_v)

    def body(k, carry):
        off0 = k * _SLABIDX
        copies = []
        for j in range(_SLAB):
            idx_sl = idx_v.at[pl.ds(off0 + j * _CH, _CH)]
            copies.append(pltpu.async_copy(
                embt_hbm.at[idx_sl], erows_v.at[pl.ds(j * _CH, _CH)], sem_g))
            copies.append(pltpu.async_copy(
                lint_hbm.at[idx_sl], lrows_v.at[pl.ds(j * _CH, _CH)], sem_g))
        for cp in copies:
            cp.wait()
        we = pltpu.async_copy(erows_v, eout_hbm.at[pl.ds(base + off0, _SLABIDX)], sem_w)
        wl = pltpu.async_copy(lrows_v, lout_hbm.at[pl.ds(base + off0, _SLABIDX)], sem_w)
        we.wait()
        wl.wait()
        return carry

    lax.fori_loop(0, _NSLAB, body, 0)


@functools.lru_cache(maxsize=1)
def _sc_gather():
    return pl.kernel(
        _sc_gather_body,
        mesh=plsc.VectorSubcoreMesh(core_axis_name="c", subcore_axis_name="s",
                                    num_cores=_NC, num_subcores=_NS),
        out_type=[
            jax.ShapeDtypeStruct((_B * _NF, _ED), jnp.float32),
            jax.ShapeDtypeStruct((_B * _NF,), jnp.float32),
        ],
        scratch_types=[
            pltpu.VMEM((_IDXW,), jnp.int32),
            pltpu.VMEM((_SLABIDX, _ED), jnp.float32),
            pltpu.VMEM((_SLABIDX,), jnp.float32),
            pltpu.SemaphoreType.DMA,
            pltpu.SemaphoreType.DMA,
        ],
        compiler_params=pltpu.CompilerParams(use_tc_tiling_on_sc=False),
    )


_NTILE = _NUM_TABLE_ROWS // 128
_TAILW = _NUM_TABLE_ROWS - _NTILE * 128
_NGRP = _NTILE // 8
_NPAIR = _NGRP // 2
_PAIR_BASE = _NPAIR // _NW
_PAIR_EXTRA = _NPAIR - _PAIR_BASE * _NW


def _cvt_body(embt_hbm, tail_hbm, out_hbm, st0, st1, ob0, ob1, si0, si1):
    c_ = lax.axis_index("c")
    s_ = lax.axis_index("s")
    wid = s_ * _NC + c_
    npair = _PAIR_BASE + (wid < _PAIR_EXTRA).astype(jnp.int32)
    pstart = wid * _PAIR_BASE + lax.min(wid, _PAIR_EXTRA)
    iota = lax.broadcasted_iota(jnp.int32, (16,), 0)

    def fire(g, st, si):
        return [pltpu.async_copy(
            embt_hbm.at[:, pl.ds((g * 8 + j) * 128, 128)], st.at[j], si)
            for j in range(8)]

    def permute(st, ob):
        for j in range(8):
            jvec = iota * 0 + j

            @plsc.parallel_loop(0, 128, unroll=16)
            def _cloop(c):
                v = plsc.load_gather(st, [jvec, iota, iota * 0 + c])
                ob[pl.ds(j * 2048 + c * 16, 16)] = v

    def pbody(k, carry):
        g0 = (pstart + k) * 2
        g1 = g0 + 1
        cp0 = fire(g0, st0, si0)
        cp1 = fire(g1, st1, si1)
        for cp in cp0:
            cp.wait()
        permute(st0, ob0)
        pltpu.sync_copy(ob0, out_hbm.at[pl.ds(g0 * 16384, 16384)])
        for cp in cp1:
            cp.wait()
        permute(st1, ob1)
        pltpu.sync_copy(ob1, out_hbm.at[pl.ds(g1 * 16384, 16384)])
        return carry

    lax.fori_loop(0, npair, pbody, 0)

    @pl.when(wid == _NW - 1)
    def _leftovers():
        gl = _NGRP - 1
        for cp in fire(gl, st0, si0):
            cp.wait()
        permute(st0, ob0)
        pltpu.sync_copy(ob0, out_hbm.at[pl.ds(gl * 16384, 16384)])
        pltpu.sync_copy(tail_hbm, ob1.at[pl.ds(0, _TAILW * _ED)])
        pltpu.sync_copy(ob1.at[pl.ds(0, _TAILW * _ED)],
                        out_hbm.at[pl.ds(_NTILE * 2048, _TAILW * _ED)])


@functools.lru_cache(maxsize=1)
def _cvt_call():
    return pl.kernel(
        _cvt_body,
        mesh=plsc.VectorSubcoreMesh(core_axis_name="c", subcore_axis_name="s",
                                    num_cores=_NC, num_subcores=_NS),
        out_type=jax.ShapeDtypeStruct((_NUM_TABLE_ROWS * _ED,), jnp.float32),
        scratch_types=[
            pltpu.VMEM((8, _ED, 128), jnp.float32),
            pltpu.VMEM((8, _ED, 128), jnp.float32),
            pltpu.VMEM((16384,), jnp.float32),
            pltpu.VMEM((16384,), jnp.float32),
            pltpu.SemaphoreType.DMA,
            pltpu.SemaphoreType.DMA,
        ],
        compiler_params=pltpu.CompilerParams(use_tc_tiling_on_sc=True,
                                             needs_layout_passes=False),
    )


_BT = 1024


def _mlp_body(emb_ref, lin_ref, w1_ref, b1_ref, g1_ref, be1_ref,
              w2_ref, b2_ref, g2_ref, be2_ref, w3t_ref, bias_ref, out_ref):
    r = float(1.0 / np.sqrt(1.0 + _EPS))
    s1 = g1_ref[...] * r
    s2 = g2_ref[...] * r
    h = jnp.dot(emb_ref[...], w1_ref[...], preferred_element_type=jnp.float32)
    h = h * s1 + (b1_ref[...] * s1 + be1_ref[...])
    h = jnp.maximum(h, 0.0)
    h = jnp.dot(h, w2_ref[...], preferred_element_type=jnp.float32)
    h = h * s2 + (b2_ref[...] * s2 + be2_ref[...])
    h = jnp.maximum(h, 0.0)
    deep = jnp.sum(h * w3t_ref[...], axis=1, keepdims=True)
    linsum = jnp.sum(lin_ref[...], axis=1, keepdims=True)
    out_ref[...] = deep + linsum + bias_ref[0, 0]


def _mlp_call(emb, lin, w1, b1, g1, be1, w2, b2, g2, be2, w3t, bias):
    full = lambda shape: pl.BlockSpec(shape, lambda i: (0, 0))
    return pl.pallas_call(
        _mlp_body,
        grid=(_B // _BT,),
        in_specs=[
            pl.BlockSpec((_BT, _NF * _ED), lambda i: (i, 0)),
            pl.BlockSpec((_BT, _NF), lambda i: (i, 0)),
            full((_NF * _ED, _H1)),
            full((1, _H1)),
            full((1, _H1)),
            full((1, _H1)),
            full((_H1, _H2)),
            full((1, _H2)),
            full((1, _H2)),
            full((1, _H2)),
            full((1, _H2)),
            full((1, 1)),
        ],
        out_specs=pl.BlockSpec((_BT, 1), lambda i: (i, 0)),
        out_shape=jax.ShapeDtypeStruct((_B, 1), jnp.float32),
    )(emb, lin, w1, b1, g1, be1, w2, b2, g2, be2, w3t, bias)


def kernel(x, emb_table, lin_table, lin_bias, W1, b1, g1, be1,
           W2, b2, g2, be2, W3, b3):
    xo = (x.astype(jnp.int32) + _OFFSETS[None, :]).reshape(-1)
    tail = emb_table[_NTILE * 128:].reshape(-1)
    table_rm = _cvt_call()(emb_table.T, tail).reshape(_NUM_TABLE_ROWS, _ED)
    erows, lrows = _sc_gather()(xo, table_rm, lin_table.reshape(-1))
    emb = erows.reshape(_B, _NF * _ED)
    lin = lrows.reshape(_B, _NF)
    bias = (lin_bias[0] + b3[0]).reshape(1, 1)
    out = _mlp_call(
        emb, lin, W1,
        b1.reshape(1, _H1), g1.reshape(1, _H1), be1.reshape(1, _H1),
        W2, b2.reshape(1, _H2), g2.reshape(1, _H2), be2.reshape(1, _H2),
        W3.reshape(1, _H2), bias)
    return out


# speedup vs baseline: 14.6403x; 1.2892x over previous
import functools

import jax
import jax.numpy as jnp
import numpy as np
from jax import lax
from jax.experimental import pallas as pl
from jax.experimental.pallas import tpu as pltpu
from jax.experimental.pallas import tpu_sc as plsc

_B = 16384
_NF = 26
_ED = 16
_H1 = 256
_H2 = 128
_EPS = 1e-5
_NUM_TABLE_ROWS = 26 * 100000
_OFFSETS = (np.arange(26, dtype=np.int32) * 100000)

_NC = 2
_NS = 16
_NW = _NC * _NS
_IDXW = (_B * _NF) // _NW
_CH = 128
_NCHUNK = _IDXW // _CH


_SLAB = 8
_SLABIDX = _SLAB * _CH
_NSLAB = _IDXW // _SLABIDX


def _sc_gather_body(xo_hbm, embt_hbm, lint_hbm, eout_hbm, lout_hbm,
                    idx_v, erows_v, lrows_v, sem_g, sem_w):
    c = lax.axis_index("c")
    s = lax.axis_index("s")
    wid = s * _NC + c
    base = wid * _IDXW
    pltpu.sync_copy(xo_hbm.at[pl.ds(base, _IDXW)], idx_v)

    def body(k, carry):
        off0 = k * _SLABIDX
        copies = []
        for j in range(_SLAB):
            idx_sl = idx_v.at[pl.ds(off0 + j * _CH, _CH)]
            copies.append(pltpu.async_copy(
                embt_hbm.at[idx_sl], erows_v.at[pl.ds(j * _CH, _CH)], sem_g))
            copies.append(pltpu.async_copy(
                lint_hbm.at[idx_sl], lrows_v.at[pl.ds(j * _CH, _CH)], sem_g))
        for cp in copies:
            cp.wait()
        we = pltpu.async_copy(erows_v, eout_hbm.at[pl.ds(base + off0, _SLABIDX)], sem_w)
        wl = pltpu.async_copy(lrows_v, lout_hbm.at[pl.ds(base + off0, _SLABIDX)], sem_w)
        we.wait()
        wl.wait()
        return carry

    lax.fori_loop(0, _NSLAB, body, 0)


@functools.lru_cache(maxsize=1)
def _sc_gather():
    return pl.kernel(
        _sc_gather_body,
        mesh=plsc.VectorSubcoreMesh(core_axis_name="c", subcore_axis_name="s",
                                    num_cores=_NC, num_subcores=_NS),
        out_type=[
            jax.ShapeDtypeStruct((_B * _NF, _ED), jnp.float32),
            jax.ShapeDtypeStruct((_B * _NF,), jnp.float32),
        ],
        scratch_types=[
            pltpu.VMEM((_IDXW,), jnp.int32),
            pltpu.VMEM((_SLABIDX, _ED), jnp.float32),
            pltpu.VMEM((_SLABIDX,), jnp.float32),
            pltpu.SemaphoreType.DMA,
            pltpu.SemaphoreType.DMA,
        ],
        compiler_params=pltpu.CompilerParams(use_tc_tiling_on_sc=False),
    )


_NTILE = _NUM_TABLE_ROWS // 128
_TAILW = _NUM_TABLE_ROWS - _NTILE * 128
_NGRP = _NTILE // 8
_NPAIR = _NGRP // 2
_PAIR_BASE = _NPAIR // _NW
_PAIR_EXTRA = _NPAIR - _PAIR_BASE * _NW


def _cvt_body(embt_hbm, tail_hbm, out_hbm, st0, st1, ob0, ob1, si0, si1):
    c_ = lax.axis_index("c")
    s_ = lax.axis_index("s")
    wid = s_ * _NC + c_
    npair = _PAIR_BASE + (wid < _PAIR_EXTRA).astype(jnp.int32)
    pstart = wid * _PAIR_BASE + lax.min(wid, _PAIR_EXTRA)
    iota = lax.broadcasted_iota(jnp.int32, (16,), 0)

    def fire(g, st, si):
        return [pltpu.async_copy(
            embt_hbm.at[:, pl.ds((g * 8 + j) * 128, 128)], st.at[j], si)
            for j in range(8)]

    i16 = iota * 16

    def permute(st, ob):
        @plsc.parallel_loop(0, 1024, unroll=8)
        def _kloop(k):
            row = k >> 3
            cc = k & 7
            j = row >> 4
            e = row & 15
            v = st[j, e, pl.ds(cc * 16, 16)]
            addr = i16 + (j * 2048 + cc * 256 + e)
            plsc.store_scatter(ob, [addr], v)

    def pbody(k, carry):
        g0 = (pstart + k) * 2
        g1 = g0 + 1
        cp0 = fire(g0, st0, si0)
        cp1 = fire(g1, st1, si1)
        for cp in cp0:
            cp.wait()
        permute(st0, ob0)
        pltpu.sync_copy(ob0, out_hbm.at[pl.ds(g0 * 16384, 16384)])
        for cp in cp1:
            cp.wait()
        permute(st1, ob1)
        pltpu.sync_copy(ob1, out_hbm.at[pl.ds(g1 * 16384, 16384)])
        return carry

    lax.fori_loop(0, npair, pbody, 0)

    @pl.when(wid == _NW - 1)
    def _leftovers():
        gl = _NGRP - 1
        for cp in fire(gl, st0, si0):
            cp.wait()
        permute(st0, ob0)
        pltpu.sync_copy(ob0, out_hbm.at[pl.ds(gl * 16384, 16384)])
        pltpu.sync_copy(tail_hbm, ob1.at[pl.ds(0, _TAILW * _ED)])
        pltpu.sync_copy(ob1.at[pl.ds(0, _TAILW * _ED)],
                        out_hbm.at[pl.ds(_NTILE * 2048, _TAILW * _ED)])


@functools.lru_cache(maxsize=1)
def _cvt_call():
    return pl.kernel(
        _cvt_body,
        mesh=plsc.VectorSubcoreMesh(core_axis_name="c", subcore_axis_name="s",
                                    num_cores=_NC, num_subcores=_NS),
        out_type=jax.ShapeDtypeStruct((_NUM_TABLE_ROWS * _ED,), jnp.float32),
        scratch_types=[
            pltpu.VMEM((8, _ED, 128), jnp.float32),
            pltpu.VMEM((8, _ED, 128), jnp.float32),
            pltpu.VMEM((16384,), jnp.float32),
            pltpu.VMEM((16384,), jnp.float32),
            pltpu.SemaphoreType.DMA,
            pltpu.SemaphoreType.DMA,
        ],
        compiler_params=pltpu.CompilerParams(use_tc_tiling_on_sc=True,
                                             needs_layout_passes=False),
    )


_BT = 1024


def _mlp_body(emb_ref, lin_ref, w1_ref, b1_ref, g1_ref, be1_ref,
              w2_ref, b2_ref, g2_ref, be2_ref, w3t_ref, bias_ref, out_ref):
    r = float(1.0 / np.sqrt(1.0 + _EPS))
    s1 = g1_ref[...] * r
    s2 = g2_ref[...] * r
    h = jnp.dot(emb_ref[...], w1_ref[...], preferred_element_type=jnp.float32)
    h = h * s1 + (b1_ref[...] * s1 + be1_ref[...])
    h = jnp.maximum(h, 0.0)
    h = jnp.dot(h, w2_ref[...], preferred_element_type=jnp.float32)
    h = h * s2 + (b2_ref[...] * s2 + be2_ref[...])
    h = jnp.maximum(h, 0.0)
    deep = jnp.sum(h * w3t_ref[...], axis=1, keepdims=True)
    linsum = jnp.sum(lin_ref[...], axis=1, keepdims=True)
    out_ref[...] = deep + linsum + bias_ref[0, 0]


def _mlp_call(emb, lin, w1, b1, g1, be1, w2, b2, g2, be2, w3t, bias):
    full = lambda shape: pl.BlockSpec(shape, lambda i: (0, 0))
    return pl.pallas_call(
        _mlp_body,
        grid=(_B // _BT,),
        in_specs=[
            pl.BlockSpec((_BT, _NF * _ED), lambda i: (i, 0)),
            pl.BlockSpec((_BT, _NF), lambda i: (i, 0)),
            full((_NF * _ED, _H1)),
            full((1, _H1)),
            full((1, _H1)),
            full((1, _H1)),
            full((_H1, _H2)),
            full((1, _H2)),
            full((1, _H2)),
            full((1, _H2)),
            full((1, _H2)),
            full((1, 1)),
        ],
        out_specs=pl.BlockSpec((_BT, 1), lambda i: (i, 0)),
        out_shape=jax.ShapeDtypeStruct((_B, 1), jnp.float32),
    )(emb, lin, w1, b1, g1, be1, w2, b2, g2, be2, w3t, bias)


def kernel(x, emb_table, lin_table, lin_bias, W1, b1, g1, be1,
           W2, b2, g2, be2, W3, b3):
    xo = (x.astype(jnp.int32) + _OFFSETS[None, :]).reshape(-1)
    tail = emb_table[_NTILE * 128:].reshape(-1)
    table_rm = _cvt_call()(emb_table.T, tail).reshape(_NUM_TABLE_ROWS, _ED)
    erows, lrows = _sc_gather()(xo, table_rm, lin_table.reshape(-1))
    emb = erows.reshape(_B, _NF * _ED)
    lin = lrows.reshape(_B, _NF)
    bias = (lin_bias[0] + b3[0]).reshape(1, 1)
    out = _mlp_call(
        emb, lin, W1,
        b1.reshape(1, _H1), g1.reshape(1, _H1), be1.reshape(1, _H1),
        W2, b2.reshape(1, _H2), g2.reshape(1, _H2), be2.reshape(1, _H2),
        W3.reshape(1, _H2), bias)
    return out


# speedup vs baseline: 15.2569x; 1.0421x over previous
import functools

import jax
import jax.numpy as jnp
import numpy as np
from jax import lax
from jax.experimental import pallas as pl
from jax.experimental.pallas import tpu as pltpu
from jax.experimental.pallas import tpu_sc as plsc

_B = 16384
_NF = 26
_ED = 16
_H1 = 256
_H2 = 128
_EPS = 1e-5
_NUM_TABLE_ROWS = 26 * 100000
_OFFSETS = (np.arange(26, dtype=np.int32) * 100000)

_NC = 2
_NS = 16
_NW = _NC * _NS
_IDXW = (_B * _NF) // _NW
_CH = 128
_NCHUNK = _IDXW // _CH


_SLAB = 8
_SLABIDX = _SLAB * _CH
_NSLAB = _IDXW // _SLABIDX


def _sc_gather_body(xo_hbm, embt_hbm, lint_hbm, eout_hbm, lout_hbm,
                    idx_v, erows_v, lrows_v, sem_g, sem_w):
    c = lax.axis_index("c")
    s = lax.axis_index("s")
    wid = s * _NC + c
    base = wid * _IDXW
    pltpu.sync_copy(xo_hbm.at[pl.ds(base, _IDXW)], idx_v)

    def body(k, carry):
        off0 = k * _SLABIDX
        copies = []
        for j in range(_SLAB):
            idx_sl = idx_v.at[pl.ds(off0 + j * _CH, _CH)]
            copies.append(pltpu.async_copy(
                embt_hbm.at[idx_sl], erows_v.at[pl.ds(j * _CH, _CH)], sem_g))
            copies.append(pltpu.async_copy(
                lint_hbm.at[idx_sl], lrows_v.at[pl.ds(j * _CH, _CH)], sem_g))
        for cp in copies:
            cp.wait()
        we = pltpu.async_copy(erows_v, eout_hbm.at[pl.ds(base + off0, _SLABIDX)], sem_w)
        wl = pltpu.async_copy(lrows_v, lout_hbm.at[pl.ds(base + off0, _SLABIDX)], sem_w)
        we.wait()
        wl.wait()
        return carry

    lax.fori_loop(0, _NSLAB, body, 0)


@functools.lru_cache(maxsize=1)
def _sc_gather():
    return pl.kernel(
        _sc_gather_body,
        mesh=plsc.VectorSubcoreMesh(core_axis_name="c", subcore_axis_name="s",
                                    num_cores=_NC, num_subcores=_NS),
        out_type=[
            jax.ShapeDtypeStruct((_B * _NF, _ED), jnp.float32),
            jax.ShapeDtypeStruct((_B * _NF,), jnp.float32),
        ],
        scratch_types=[
            pltpu.VMEM((_IDXW,), jnp.int32),
            pltpu.VMEM((_SLABIDX, _ED), jnp.float32),
            pltpu.VMEM((_SLABIDX,), jnp.float32),
            pltpu.SemaphoreType.DMA,
            pltpu.SemaphoreType.DMA,
        ],
        compiler_params=pltpu.CompilerParams(use_tc_tiling_on_sc=False),
    )


_NTILE = _NUM_TABLE_ROWS // 128
_TAILW = _NUM_TABLE_ROWS - _NTILE * 128
_NGRP = _NTILE // 8
_NPAIR = _NGRP // 2
_PAIR_BASE = _NPAIR // _NW
_PAIR_EXTRA = _NPAIR - _PAIR_BASE * _NW


def _cvt_body(embt_hbm, tail_hbm, out_hbm, st0, st1, ob0, ob1, si0, si1):
    c_ = lax.axis_index("c")
    s_ = lax.axis_index("s")
    wid = s_ * _NC + c_
    npair = _PAIR_BASE + (wid < _PAIR_EXTRA).astype(jnp.int32)
    pstart = wid * _PAIR_BASE + lax.min(wid, _PAIR_EXTRA)
    iota = lax.broadcasted_iota(jnp.int32, (16,), 0)

    def fire(g, st, si):
        return [pltpu.async_copy(
            embt_hbm.at[:, pl.ds((g * 8 + j) * 128, 128)], st.at[j], si)
            for j in range(8)]

    i16 = iota * 16

    def permute(st, ob):
        @plsc.parallel_loop(0, 1024, unroll=8)
        def _kloop(k):
            row = k >> 3
            cc = k & 7
            j = row >> 4
            e = row & 15
            v = st[j, e, pl.ds(cc * 16, 16)]
            addr = i16 + (j * 2048 + cc * 256 + e)
            plsc.store_scatter(ob, [addr], v)

    def pbody(k, carry):
        g0 = (pstart + k) * 2
        g1 = g0 + 1
        cp0 = fire(g0, st0, si0)
        cp1 = fire(g1, st1, si1)
        for cp in cp0:
            cp.wait()
        permute(st0, ob0)
        w0 = pltpu.async_copy(ob0, out_hbm.at[pl.ds(g0 * 16384, 16384)], si0)
        for cp in cp1:
            cp.wait()
        permute(st1, ob1)
        w1 = pltpu.async_copy(ob1, out_hbm.at[pl.ds(g1 * 16384, 16384)], si1)
        w0.wait()
        w1.wait()
        return carry

    lax.fori_loop(0, npair, pbody, 0)

    @pl.when(wid == _NW - 1)
    def _leftovers():
        gl = _NGRP - 1
        for cp in fire(gl, st0, si0):
            cp.wait()
        permute(st0, ob0)
        pltpu.sync_copy(ob0, out_hbm.at[pl.ds(gl * 16384, 16384)])
        pltpu.sync_copy(tail_hbm, ob1.at[pl.ds(0, _TAILW * _ED)])
        pltpu.sync_copy(ob1.at[pl.ds(0, _TAILW * _ED)],
                        out_hbm.at[pl.ds(_NTILE * 2048, _TAILW * _ED)])


@functools.lru_cache(maxsize=1)
def _cvt_call():
    return pl.kernel(
        _cvt_body,
        mesh=plsc.VectorSubcoreMesh(core_axis_name="c", subcore_axis_name="s",
                                    num_cores=_NC, num_subcores=_NS),
        out_type=jax.ShapeDtypeStruct((_NUM_TABLE_ROWS * _ED,), jnp.float32),
        scratch_types=[
            pltpu.VMEM((8, _ED, 128), jnp.float32),
            pltpu.VMEM((8, _ED, 128), jnp.float32),
            pltpu.VMEM((16384,), jnp.float32),
            pltpu.VMEM((16384,), jnp.float32),
            pltpu.SemaphoreType.DMA,
            pltpu.SemaphoreType.DMA,
        ],
        compiler_params=pltpu.CompilerParams(use_tc_tiling_on_sc=True,
                                             needs_layout_passes=False),
    )


_BT = 1024


def _mlp_body(emb_ref, lin_ref, w1_ref, b1_ref, g1_ref, be1_ref,
              w2_ref, b2_ref, g2_ref, be2_ref, w3t_ref, bias_ref, out_ref):
    r = float(1.0 / np.sqrt(1.0 + _EPS))
    s1 = g1_ref[...] * r
    s2 = g2_ref[...] * r
    h = jnp.dot(emb_ref[...], w1_ref[...], preferred_element_type=jnp.float32)
    h = h * s1 + (b1_ref[...] * s1 + be1_ref[...])
    h = jnp.maximum(h, 0.0)
    h = jnp.dot(h, w2_ref[...], preferred_element_type=jnp.float32)
    h = h * s2 + (b2_ref[...] * s2 + be2_ref[...])
    h = jnp.maximum(h, 0.0)
    deep = jnp.sum(h * w3t_ref[...], axis=1, keepdims=True)
    linsum = jnp.sum(lin_ref[...], axis=1, keepdims=True)
    out_ref[...] = deep + linsum + bias_ref[0, 0]


def _mlp_call(emb, lin, w1, b1, g1, be1, w2, b2, g2, be2, w3t, bias):
    full = lambda shape: pl.BlockSpec(shape, lambda i: (0, 0))
    return pl.pallas_call(
        _mlp_body,
        grid=(_B // _BT,),
        in_specs=[
            pl.BlockSpec((_BT, _NF * _ED), lambda i: (i, 0)),
            pl.BlockSpec((_BT, _NF), lambda i: (i, 0)),
            full((_NF * _ED, _H1)),
            full((1, _H1)),
            full((1, _H1)),
            full((1, _H1)),
            full((_H1, _H2)),
            full((1, _H2)),
            full((1, _H2)),
            full((1, _H2)),
            full((1, _H2)),
            full((1, 1)),
        ],
        out_specs=pl.BlockSpec((_BT, 1), lambda i: (i, 0)),
        out_shape=jax.ShapeDtypeStruct((_B, 1), jnp.float32),
    )(emb, lin, w1, b1, g1, be1, w2, b2, g2, be2, w3t, bias)


def kernel(x, emb_table, lin_table, lin_bias, W1, b1, g1, be1,
           W2, b2, g2, be2, W3, b3):
    xo = (x.astype(jnp.int32) + _OFFSETS[None, :]).reshape(-1)
    tail = emb_table[_NTILE * 128:].reshape(-1)
    table_rm = _cvt_call()(emb_table.T, tail).reshape(_NUM_TABLE_ROWS, _ED)
    erows, lrows = _sc_gather()(xo, table_rm, lin_table.reshape(-1))
    emb = erows.reshape(_B, _NF * _ED)
    lin = lrows.reshape(_B, _NF)
    bias = (lin_bias[0] + b3[0]).reshape(1, 1)
    out = _mlp_call(
        emb, lin, W1,
        b1.reshape(1, _H1), g1.reshape(1, _H1), be1.reshape(1, _H1),
        W2, b2.reshape(1, _H2), g2.reshape(1, _H2), be2.reshape(1, _H2),
        W3.reshape(1, _H2), bias)
    return out


# speedup vs baseline: 16.9804x; 1.1130x over previous
import functools

import jax
import jax.numpy as jnp
import numpy as np
from jax import lax
from jax.experimental import pallas as pl
from jax.experimental.pallas import tpu as pltpu
from jax.experimental.pallas import tpu_sc as plsc

_B = 16384
_NF = 26
_ED = 16
_H1 = 256
_H2 = 128
_EPS = 1e-5
_NUM_TABLE_ROWS = 26 * 100000
_OFFSETS = (np.arange(26, dtype=np.int32) * 100000)

_NC = 2
_NS = 16
_NW = _NC * _NS
_IDXW = (_B * _NF) // _NW
_CH = 128
_NCHUNK = _IDXW // _CH


_SLAB = 8
_SLABIDX = _SLAB * _CH
_NSLAB = _IDXW // _SLABIDX


def _sc_gather_body(xo_hbm, embt_hbm, lint_hbm, eout_hbm, lout_hbm,
                    idx_v, erows_v, lrows_v, sem_g, sem_w):
    c = lax.axis_index("c")
    s = lax.axis_index("s")
    wid = s * _NC + c
    base = wid * _IDXW
    pltpu.sync_copy(xo_hbm.at[pl.ds(base, _IDXW)], idx_v)

    def body(k, carry):
        off0 = k * _SLABIDX
        copies = []
        for j in range(_SLAB):
            idx_sl = idx_v.at[pl.ds(off0 + j * _CH, _CH)]
            copies.append(pltpu.async_copy(
                embt_hbm.at[idx_sl], erows_v.at[pl.ds(j * _CH, _CH)], sem_g))
            copies.append(pltpu.async_copy(
                lint_hbm.at[idx_sl], lrows_v.at[pl.ds(j * _CH, _CH)], sem_g))
        for cp in copies:
            cp.wait()
        we = pltpu.async_copy(erows_v, eout_hbm.at[pl.ds(base + off0, _SLABIDX)], sem_w)
        wl = pltpu.async_copy(lrows_v, lout_hbm.at[pl.ds(base + off0, _SLABIDX)], sem_w)
        we.wait()
        wl.wait()
        return carry

    lax.fori_loop(0, _NSLAB, body, 0)


@functools.lru_cache(maxsize=1)
def _sc_gather():
    return pl.kernel(
        _sc_gather_body,
        mesh=plsc.VectorSubcoreMesh(core_axis_name="c", subcore_axis_name="s",
                                    num_cores=_NC, num_subcores=_NS),
        out_type=[
            jax.ShapeDtypeStruct((_B * _NF, _ED), jnp.float32),
            jax.ShapeDtypeStruct((_B * _NF,), jnp.float32),
        ],
        scratch_types=[
            pltpu.VMEM((_IDXW,), jnp.int32),
            pltpu.VMEM((_SLABIDX, _ED), jnp.float32),
            pltpu.VMEM((_SLABIDX,), jnp.float32),
            pltpu.SemaphoreType.DMA,
            pltpu.SemaphoreType.DMA,
        ],
        compiler_params=pltpu.CompilerParams(use_tc_tiling_on_sc=False),
    )


_NTILE = _NUM_TABLE_ROWS // 128
_TAILW = _NUM_TABLE_ROWS - _NTILE * 128
_NGRP = _NTILE // 8
_NPAIR = _NGRP // 2
_PAIR_BASE = _NPAIR // _NW
_PAIR_EXTRA = _NPAIR - _PAIR_BASE * _NW


def _cvt_body(embt_hbm, tail_hbm, out_hbm, st0, st1, ob0, ob1, si0, si1):
    c_ = lax.axis_index("c")
    s_ = lax.axis_index("s")
    wid = s_ * _NC + c_
    npair = _PAIR_BASE + (wid < _PAIR_EXTRA).astype(jnp.int32)
    pstart = wid * _PAIR_BASE + lax.min(wid, _PAIR_EXTRA)
    iota = lax.broadcasted_iota(jnp.int32, (16,), 0)

    def fire(g, st, si):
        return [pltpu.async_copy(
            embt_hbm.at[:, pl.ds((g * 8 + j) * 128, 128)], st.at[j], si)
            for j in range(8)]

    i16 = iota * 16

    def permute(st, ob):
        @plsc.parallel_loop(0, 1024, unroll=16)
        def _kloop(k):
            row = k >> 3
            cc = k & 7
            j = row >> 4
            e = row & 15
            v = st[j, e, pl.ds(cc * 16, 16)]
            addr = i16 + (j * 2048 + cc * 256 + e)
            plsc.store_scatter(ob, [addr], v)

    def pbody(k, carry):
        g0 = (pstart + k) * 2
        g1 = g0 + 1
        cp0 = fire(g0, st0, si0)
        cp1 = fire(g1, st1, si1)
        for cp in cp0:
            cp.wait()
        permute(st0, ob0)
        w0 = pltpu.async_copy(ob0, out_hbm.at[pl.ds(g0 * 16384, 16384)], si0)
        for cp in cp1:
            cp.wait()
        permute(st1, ob1)
        w1 = pltpu.async_copy(ob1, out_hbm.at[pl.ds(g1 * 16384, 16384)], si1)
        w0.wait()
        w1.wait()
        return carry

    lax.fori_loop(0, npair, pbody, 0)

    @pl.when(wid == _NW - 1)
    def _leftovers():
        gl = _NGRP - 1
        for cp in fire(gl, st0, si0):
            cp.wait()
        permute(st0, ob0)
        pltpu.sync_copy(ob0, out_hbm.at[pl.ds(gl * 16384, 16384)])
        pltpu.sync_copy(tail_hbm, ob1.at[pl.ds(0, _TAILW * _ED)])
        pltpu.sync_copy(ob1.at[pl.ds(0, _TAILW * _ED)],
                        out_hbm.at[pl.ds(_NTILE * 2048, _TAILW * _ED)])


@functools.lru_cache(maxsize=1)
def _cvt_call():
    return pl.kernel(
        _cvt_body,
        mesh=plsc.VectorSubcoreMesh(core_axis_name="c", subcore_axis_name="s",
                                    num_cores=_NC, num_subcores=_NS),
        out_type=jax.ShapeDtypeStruct((_NUM_TABLE_ROWS * _ED,), jnp.float32),
        scratch_types=[
            pltpu.VMEM((8, _ED, 128), jnp.float32),
            pltpu.VMEM((8, _ED, 128), jnp.float32),
            pltpu.VMEM((16384,), jnp.float32),
            pltpu.VMEM((16384,), jnp.float32),
            pltpu.SemaphoreType.DMA,
            pltpu.SemaphoreType.DMA,
        ],
        compiler_params=pltpu.CompilerParams(use_tc_tiling_on_sc=True,
                                             needs_layout_passes=False),
    )


_BT = 1024


def _mlp_body(emb_ref, lin_ref, w1_ref, b1_ref, g1_ref, be1_ref,
              w2_ref, b2_ref, g2_ref, be2_ref, w3t_ref, bias_ref, out_ref):
    r = float(1.0 / np.sqrt(1.0 + _EPS))
    s1 = g1_ref[...] * r
    s2 = g2_ref[...] * r
    h = jnp.dot(emb_ref[...], w1_ref[...], preferred_element_type=jnp.float32)
    h = h * s1 + (b1_ref[...] * s1 + be1_ref[...])
    h = jnp.maximum(h, 0.0)
    h = jnp.dot(h, w2_ref[...], preferred_element_type=jnp.float32)
    h = h * s2 + (b2_ref[...] * s2 + be2_ref[...])
    h = jnp.maximum(h, 0.0)
    deep = jnp.sum(h * w3t_ref[...], axis=1, keepdims=True)
    linsum = jnp.sum(lin_ref[...], axis=1, keepdims=True)
    out_ref[...] = deep + linsum + bias_ref[0, 0]


def _mlp_call(emb, lin, w1, b1, g1, be1, w2, b2, g2, be2, w3t, bias):
    full = lambda shape: pl.BlockSpec(shape, lambda i: (0, 0))
    return pl.pallas_call(
        _mlp_body,
        grid=(_B // _BT,),
        in_specs=[
            pl.BlockSpec((_BT, _NF * _ED), lambda i: (i, 0)),
            pl.BlockSpec((_BT, _NF), lambda i: (i, 0)),
            full((_NF * _ED, _H1)),
            full((1, _H1)),
            full((1, _H1)),
            full((1, _H1)),
            full((_H1, _H2)),
            full((1, _H2)),
            full((1, _H2)),
            full((1, _H2)),
            full((1, _H2)),
            full((1, 1)),
        ],
        out_specs=pl.BlockSpec((_BT, 1), lambda i: (i, 0)),
        out_shape=jax.ShapeDtypeStruct((_B, 1), jnp.float32),
    )(emb, lin, w1, b1, g1, be1, w2, b2, g2, be2, w3t, bias)


def kernel(x, emb_table, lin_table, lin_bias, W1, b1, g1, be1,
           W2, b2, g2, be2, W3, b3):
    xo = (x.astype(jnp.int32) + _OFFSETS[None, :]).reshape(-1)
    tail = emb_table[_NTILE * 128:].reshape(-1)
    table_rm = _cvt_call()(emb_table.T, tail).reshape(_NUM_TABLE_ROWS, _ED)
    erows, lrows = _sc_gather()(xo, table_rm, lin_table.reshape(-1))
    emb = erows.reshape(_B, _NF * _ED)
    lin = lrows.reshape(_B, _NF)
    bias = (lin_bias[0] + b3[0]).reshape(1, 1)
    out = _mlp_call(
        emb, lin, W1,
        b1.reshape(1, _H1), g1.reshape(1, _H1), be1.reshape(1, _H1),
        W2, b2.reshape(1, _H2), g2.reshape(1, _H2), be2.reshape(1, _H2),
        W3.reshape(1, _H2), bias)
    return out
